# baseline (speedup 1.0000x reference)
import functools

import numpy as np
import jax
import jax.numpy as jnp
from jax import lax
from jax.experimental import pallas as pl
from jax.experimental.pallas import tpu as pltpu
from jax.experimental.pallas import tpu_sc as plsc


def _positional_encoding_np(seq_length: int, d: int, n: float = 10000.0) -> np.ndarray:
    k = np.arange(seq_length, dtype=np.float32)[:, None]
    i = np.arange(d // 2, dtype=np.float32)[None, :]
    denominator = np.power(np.float32(n), 2.0 * i / d).astype(np.float32)
    p = np.zeros((seq_length, d), dtype=np.float32)
    p[:, 0::2] = np.sin(k / denominator)
    p[:, 1::2] = np.cos(k / denominator)
    return p


@functools.lru_cache(maxsize=None)
def _build_sc_kernel(B: int, S: int, D: int, V: int):
    info = plsc.get_sparse_core_info()
    nc, ns = info.num_cores, info.num_subcores
    nw = nc * ns
    assert B % (nw * 2) == 0 and D == 64
    spw = B // nw
    sp = -(-S // 16) * 16

    mesh = plsc.VectorSubcoreMesh(core_axis_name="c", subcore_axis_name="s")

    @functools.partial(
        pl.kernel,
        mesh=mesh,
        out_type=jax.ShapeDtypeStruct((B, S, D), jnp.float32),
        scratch_types=[
            pltpu.VMEM((2, S), jnp.int32),
            pltpu.VMEM((sp,), jnp.int32),
            pltpu.VMEM((sp,), jnp.int32),
            pltpu.VMEM((sp * 16,), jnp.int32),
            pltpu.VMEM((sp * 16,), jnp.int32),
            pltpu.VMEM((sp, 2 * D), jnp.float32),
            pltpu.VMEM((sp, 2 * D), jnp.float32),
            pltpu.VMEM((sp, D), jnp.float32),
            pltpu.VMEM((sp, D), jnp.float32),
            pltpu.VMEM((S * D,), jnp.float32),
        ]
        + [pltpu.SemaphoreType.DMA] * 4,
    )
    def k(x_hbm, tab_hbm, pe_hbm, out_hbm, raw_v, idx0_v, idx1_v, par0_v,
          par1_v, rows0_v, rows1_v, out0_v, out1_v, pe_v, *sems):
        idx = (idx0_v, idx1_v)
        par = (par0_v, par1_v)
        rows = (rows0_v, rows1_v)
        outp = (out0_v, out1_v)
        sg = sems[:2]
        ss = sems[2:]
        wid = lax.axis_index("s") * nc + lax.axis_index("c")
        pltpu.sync_copy(pe_hbm, pe_v)
        vmax = V // 2 - 1

        def prep_pair(p):
            pltpu.sync_copy(x_hbm.at[pl.ds(wid * spw + 2 * p, 2)], raw_v)
            zero16 = jnp.zeros((16,), jnp.int32)
            for j in range(2):
                cs = [t * 16 for t in range(S // 16)]
                if S % 16:
                    idx[j][pl.ds(sp - 16, 16)] = zero16
                    cs.append(S - 16)
                for c in cs:
                    sl = pl.ds(c, 16)
                    r = raw_v[j, sl]
                    idx[j][sl] = lax.min(lax.shift_right_logical(r, 1), vmax)
                    pc = r & 1
                    for q in range(16):
                        par[j][pl.ds((c + q) * 16, 16)] = jnp.take(
                            pc, jnp.full((16,), q, jnp.int32)
                        )

        def gather_start(b):
            pltpu.async_copy(tab_hbm.at[idx[b]], rows[b], sg[b])

        def gather_wait(b):
            pltpu.make_async_copy(tab_hbm.at[idx[b]], rows[b], sg[b]).wait()

        def store_copy(i, b):
            return pltpu.make_async_copy(
                outp[b].at[pl.ds(0, S)], out_hbm.at[wid * spw + i], ss[b]
            )

        prep_pair(0)
        gather_start(0)
        gather_start(1)

        def add_seq(b):
            def add_row(r, c2):
                m = par[b][pl.ds(r * 16, 16)] == 1
                for t in range(D // 16):
                    lo = rows[b][r, pl.ds(t * 16, 16)]
                    hi = rows[b][r, pl.ds(D + t * 16, 16)]
                    outp[b][r, pl.ds(t * 16, 16)] = (
                        jnp.where(m, hi, lo)
                        + pe_v[pl.ds(r * D + t * 16, 16)]
                    )
                return c2

            lax.fori_loop(0, S, add_row, 0, unroll=2)

        def group(g, carry):
            i0 = g * 2
            gather_wait(0)

            @pl.when(i0 >= 2)
            def _drain_s0():
                store_copy(i0 - 2, 0).wait()

            add_seq(0)
            store_copy(i0, 0).start()
            gather_wait(1)

            @pl.when(i0 >= 2)
            def _drain_s1():
                store_copy(i0 - 1, 1).wait()

            add_seq(1)
            store_copy(i0 + 1, 1).start()

            @pl.when(i0 + 2 < spw)
            def _prep_next():
                prep_pair(g + 1)
                gather_start(0)
                gather_start(1)
            return carry

        lax.fori_loop(0, spw // 2, group, 0)
        store_copy(spw - 2, 0).wait()
        store_copy(spw - 1, 1).wait()

    return k


def kernel(x, emb_table):
    B, S = x.shape
    V, D = emb_table.shape
    pe = jnp.asarray(_positional_encoding_np(S, D))
    tab2 = emb_table.reshape(V // 2, 2 * D)
    return _build_sc_kernel(B, S, D, V)(
        x.astype(jnp.int32), tab2, pe.reshape(-1)
    )


# speedup vs baseline: 2.0709x; 2.0709x over previous
import functools

import numpy as np
import jax
import jax.numpy as jnp
from jax import lax
from jax.experimental import pallas as pl
from jax.experimental.pallas import tpu as pltpu
from jax.experimental.pallas import tpu_sc as plsc

_NBUF = 4


def _positional_encoding_np(seq_length: int, d: int, n: float = 10000.0) -> np.ndarray:
    k = np.arange(seq_length, dtype=np.float32)[:, None]
    i = np.arange(d // 2, dtype=np.float32)[None, :]
    denominator = np.power(np.float32(n), 2.0 * i / d).astype(np.float32)
    p = np.zeros((seq_length, d), dtype=np.float32)
    p[:, 0::2] = np.sin(k / denominator)
    p[:, 1::2] = np.cos(k / denominator)
    return p


@functools.lru_cache(maxsize=None)
def _build_sc_kernel(B: int, S: int, D: int):
    info = plsc.get_sparse_core_info()
    nc, ns = info.num_cores, info.num_subcores
    nw = nc * ns
    assert B % (nw * _NBUF) == 0 and D % 16 == 0 and (S * 4) % 8 == 0
    spw = B // nw

    mesh = plsc.VectorSubcoreMesh(core_axis_name="c", subcore_axis_name="s")

    @functools.partial(
        pl.kernel,
        mesh=mesh,
        compiler_params=pltpu.CompilerParams(use_tc_tiling_on_sc=False),
        out_type=jax.ShapeDtypeStruct((B, S, D), jnp.float32),
        scratch_types=[
            pltpu.VMEM((spw, S), jnp.int32),
            pltpu.VMEM((_NBUF, S, D), jnp.float32),
            pltpu.VMEM((S, D), jnp.float32),
        ]
        + [pltpu.SemaphoreType.DMA] * (2 * _NBUF),
    )
    def k(x_hbm, tab_hbm, pe_hbm, out_hbm, idx_v, rows_v, pe_v, *sems):
        sg, ss = sems[:_NBUF], sems[_NBUF:]
        wid = lax.axis_index("s") * nc + lax.axis_index("c")
        pltpu.sync_copy(x_hbm.at[pl.ds(wid * spw, spw)], idx_v)
        pltpu.sync_copy(pe_hbm, pe_v)

        def gather_start(i, b):
            pltpu.async_copy(tab_hbm.at[idx_v.at[i]], rows_v.at[b], sg[b])

        def store_copy(i, b):
            return pltpu.make_async_copy(
                rows_v.at[b], out_hbm.at[wid * spw + i], ss[b]
            )

        gather_start(0, 0)

        def group(g, carry):
            for bk in range(_NBUF):
                i = g * _NBUF + bk
                b = bk
                b1 = (bk + 1) % _NBUF

                @pl.when(i + 1 < spw)
                def _start_next():
                    @pl.when(i >= _NBUF - 1)
                    def _drain_store():
                        store_copy(i + 1 - _NBUF, b1).wait()

                    gather_start(i + 1, b1)

                pltpu.make_async_copy(
                    tab_hbm.at[idx_v.at[i]], rows_v.at[b], sg[b]
                ).wait()

                def add_row(r, c2):
                    for t in range(D // 16):
                        sl = pl.ds(t * 16, 16)
                        rows_v[b, r, sl] = rows_v[b, r, sl] + pe_v[r, sl]
                    return c2

                lax.fori_loop(0, S, add_row, 0, unroll=2)
                store_copy(i, b).start()
            return carry

        lax.fori_loop(0, spw // _NBUF, group, 0)
        for bk in range(_NBUF):
            store_copy(spw - _NBUF + bk, bk).wait()

    return k


def kernel(x, emb_table):
    B, S = x.shape
    V, D = emb_table.shape
    pe = jnp.asarray(_positional_encoding_np(S, D))
    return _build_sc_kernel(B, S, D)(x.astype(jnp.int32), emb_table, pe)


# speedup vs baseline: 2.1845x; 1.0549x over previous
import functools

import numpy as np
import jax
import jax.numpy as jnp
from jax import lax
from jax.experimental import pallas as pl
from jax.experimental.pallas import tpu as pltpu
from jax.experimental.pallas import tpu_sc as plsc

_NBUF = 4


def _positional_encoding_np(seq_length: int, d: int, n: float = 10000.0) -> np.ndarray:
    k = np.arange(seq_length, dtype=np.float32)[:, None]
    i = np.arange(d // 2, dtype=np.float32)[None, :]
    denominator = np.power(np.float32(n), 2.0 * i / d).astype(np.float32)
    p = np.zeros((seq_length, d), dtype=np.float32)
    p[:, 0::2] = np.sin(k / denominator)
    p[:, 1::2] = np.cos(k / denominator)
    return p


@functools.lru_cache(maxsize=None)
def _build_sc_kernel(B: int, S: int, D: int):
    info = plsc.get_sparse_core_info()
    nc, ns = info.num_cores, info.num_subcores
    nw = nc * ns
    assert B % (nw * _NBUF) == 0 and D % 16 == 0 and (S * 4) % 8 == 0
    spw = B // nw

    mesh = plsc.VectorSubcoreMesh(core_axis_name="c", subcore_axis_name="s")

    @functools.partial(
        pl.kernel,
        mesh=mesh,
        compiler_params=pltpu.CompilerParams(use_tc_tiling_on_sc=False),
        out_type=jax.ShapeDtypeStruct((B, S, D), jnp.float32),
        scratch_types=[
            pltpu.VMEM((spw, S), jnp.int32),
            pltpu.VMEM((_NBUF, S, D), jnp.float32),
            pltpu.VMEM((S, D), jnp.float32),
        ]
        + [pltpu.SemaphoreType.DMA] * (2 * _NBUF),
    )
    def k(x_hbm, tab_hbm, pe_hbm, out_hbm, idx_v, rows_v, pe_v, *sems):
        sg, ss = sems[:_NBUF], sems[_NBUF:]
        wid = lax.axis_index("s") * nc + lax.axis_index("c")
        pltpu.sync_copy(x_hbm.at[pl.ds(wid * spw, spw)], idx_v)
        pltpu.sync_copy(pe_hbm, pe_v)

        def gather_start(i, b):
            pltpu.async_copy(tab_hbm.at[idx_v.at[i]], rows_v.at[b], sg[b])

        def store_copy(i, b):
            return pltpu.make_async_copy(
                rows_v.at[b], out_hbm.at[wid * spw + i], ss[b]
            )

        gather_start(0, 0)
        gather_start(1, 1)

        def group(g, carry):
            for k in range(_NBUF // 2):
                i = g * _NBUF + 2 * k
                b = 2 * k
                b1 = 2 * k + 1
                b2 = (b + 2) % _NBUF
                b3 = (b + 3) % _NBUF

                @pl.when(i + 2 < spw)
                def _start_n2():
                    @pl.when(i >= 2)
                    def _drain_s2():
                        store_copy(i - 2, b2).wait()

                    gather_start(i + 2, b2)

                @pl.when(i + 3 < spw)
                def _start_n3():
                    @pl.when(i >= 1)
                    def _drain_s3():
                        store_copy(i - 1, b3).wait()

                    gather_start(i + 3, b3)

                pltpu.make_async_copy(
                    tab_hbm.at[idx_v.at[i]], rows_v.at[b], sg[b]
                ).wait()
                pltpu.make_async_copy(
                    tab_hbm.at[idx_v.at[i + 1]], rows_v.at[b1], sg[b1]
                ).wait()

                def add_row(r, c2):
                    for t in range(D // 16):
                        sl = pl.ds(t * 16, 16)
                        pe16 = pe_v[r, sl]
                        rows_v[b, r, sl] = rows_v[b, r, sl] + pe16
                        rows_v[b1, r, sl] = rows_v[b1, r, sl] + pe16
                    return c2

                lax.fori_loop(0, S, add_row, 0, unroll=2)
                store_copy(i, b).start()
                store_copy(i + 1, b1).start()
            return carry

        lax.fori_loop(0, spw // _NBUF, group, 0)
        for bk in range(_NBUF):
            store_copy(spw - _NBUF + bk, bk).wait()

    return k


def kernel(x, emb_table):
    B, S = x.shape
    V, D = emb_table.shape
    pe = jnp.asarray(_positional_encoding_np(S, D))
    return _build_sc_kernel(B, S, D)(x.astype(jnp.int32), emb_table, pe)


# speedup vs baseline: 2.1932x; 1.0040x over previous
import functools

import numpy as np
import jax
import jax.numpy as jnp
from jax import lax
from jax.experimental import pallas as pl
from jax.experimental.pallas import tpu as pltpu
from jax.experimental.pallas import tpu_sc as plsc

_NBUF = 4


def _positional_encoding_np(seq_length: int, d: int, n: float = 10000.0) -> np.ndarray:
    k = np.arange(seq_length, dtype=np.float32)[:, None]
    i = np.arange(d // 2, dtype=np.float32)[None, :]
    denominator = np.power(np.float32(n), 2.0 * i / d).astype(np.float32)
    p = np.zeros((seq_length, d), dtype=np.float32)
    p[:, 0::2] = np.sin(k / denominator)
    p[:, 1::2] = np.cos(k / denominator)
    return p


@functools.lru_cache(maxsize=None)
def _build_sc_kernel(B: int, S: int, D: int):
    info = plsc.get_sparse_core_info()
    nc, ns = info.num_cores, info.num_subcores
    nw = nc * ns
    assert B % (nw * _NBUF) == 0 and D % 16 == 0 and (S * 4) % 8 == 0
    spw = B // nw

    mesh = plsc.VectorSubcoreMesh(core_axis_name="c", subcore_axis_name="s")

    @functools.partial(
        pl.kernel,
        mesh=mesh,
        compiler_params=pltpu.CompilerParams(use_tc_tiling_on_sc=False),
        out_type=jax.ShapeDtypeStruct((B, S, D), jnp.float32),
        scratch_types=[
            pltpu.VMEM((spw, S), jnp.int32),
            pltpu.VMEM((_NBUF, S, D), jnp.float32),
            pltpu.VMEM((S, D), jnp.float32),
        ]
        + [pltpu.SemaphoreType.DMA] * (2 * _NBUF),
    )
    def k(x_hbm, tab_hbm, pe_hbm, out_hbm, idx_v, rows_v, pe_v, *sems):
        sg, ss = sems[:_NBUF], sems[_NBUF:]
        wid = lax.axis_index("s") * nc + lax.axis_index("c")
        pltpu.sync_copy(x_hbm.at[pl.ds(wid * spw, spw)], idx_v)
        pltpu.sync_copy(pe_hbm, pe_v)

        def gather_start(i, b):
            pltpu.async_copy(tab_hbm.at[idx_v.at[i]], rows_v.at[b], sg[b])

        def store_copy(i, b):
            return pltpu.make_async_copy(
                rows_v.at[b], out_hbm.at[wid * spw + i], ss[b]
            )

        gather_start(0, 0)
        gather_start(1, 1)

        def group(g, carry):
            for k in range(_NBUF // 2):
                i = g * _NBUF + 2 * k
                b = 2 * k
                b1 = 2 * k + 1
                b2 = (b + 2) % _NBUF
                b3 = (b + 3) % _NBUF

                @pl.when(i + 2 < spw)
                def _start_n2():
                    @pl.when(i >= 2)
                    def _drain_s2():
                        store_copy(i - 2, b2).wait()

                    gather_start(i + 2, b2)

                @pl.when(i + 3 < spw)
                def _start_n3():
                    @pl.when(i >= 1)
                    def _drain_s3():
                        store_copy(i - 1, b3).wait()

                    gather_start(i + 3, b3)

                pltpu.make_async_copy(
                    tab_hbm.at[idx_v.at[i]], rows_v.at[b], sg[b]
                ).wait()
                pltpu.make_async_copy(
                    tab_hbm.at[idx_v.at[i + 1]], rows_v.at[b1], sg[b1]
                ).wait()

                def add_row(r, c2):
                    for t in range(D // 16):
                        sl = pl.ds(t * 16, 16)
                        pe16 = pe_v[r, sl]
                        rows_v[b, r, sl] = rows_v[b, r, sl] + pe16
                        rows_v[b1, r, sl] = rows_v[b1, r, sl] + pe16
                    return c2

                lax.fori_loop(0, S, add_row, 0, unroll=4)
                store_copy(i, b).start()
                store_copy(i + 1, b1).start()
            return carry

        lax.fori_loop(0, spw // _NBUF, group, 0)
        for bk in range(_NBUF):
            store_copy(spw - _NBUF + bk, bk).wait()

    return k


def kernel(x, emb_table):
    B, S = x.shape
    V, D = emb_table.shape
    pe = jnp.asarray(_positional_encoding_np(S, D))
    return _build_sc_kernel(B, S, D)(x.astype(jnp.int32), emb_table, pe)
